# baseline (speedup 1.0000x reference)
import functools

import jax
import jax.numpy as jnp
from jax import lax
from jax.experimental import pallas as pl
from jax.experimental.pallas import tpu as pltpu
from jax.experimental.pallas import tpu_sc as plsc

_NC = 2
_NS = 16
_NW = _NC * _NS
_LANES = 16
_CH = 128


def _make_seg_sum(N, H, B):
    segs_per_w = B // _NW
    lanes_per_row = H // _LANES
    mesh = plsc.VectorSubcoreMesh(core_axis_name="c", subcore_axis_name="s")

    @functools.partial(
        pl.kernel,
        mesh=mesh,
        out_type=jax.ShapeDtypeStruct((B * H,), jnp.float32),
        scratch_types=[
            pltpu.VMEM((16,), jnp.int32),
            pltpu.VMEM((_CH, H), jnp.float32),
            pltpu.VMEM((_CH, H), jnp.float32),
            pltpu.VMEM(((segs_per_w + 1) * H,), jnp.float32),
            pltpu.SemaphoreType.DMA,
            pltpu.SemaphoreType.DMA,
        ],
    )
    def seg_sum(emb_hbm, tbl_hbm, out_hbm, tblv, buf0, buf1, acc, sem0, sem1):
        wid = lax.axis_index("s") * _NC + lax.axis_index("c")
        pltpu.sync_copy(tbl_hbm.at[pl.ds(wid * 16, 16)], tblv)
        vec = tblv[...]
        bounds = [vec[j] for j in range(segs_per_w + 1)]
        b0 = bounds[0]
        bL = bounds[segs_per_w]

        zero = jnp.zeros((16,), jnp.float32)
        for i in range((segs_per_w + 1) * H // 16):
            acc[pl.ds(i * 16, 16)] = zero

        a0 = (b0 // 8) * 8
        nch = (bL - a0 + _CH - 1) // _CH

        def off(k):
            return pl.multiple_of(jnp.minimum(a0 + k * _CH, N - _CH), 8)

        def accum(buf, k):
            lo = a0 + k * _CH
            o = off(k)

            def row_body(r, carry):
                g = o + r
                valid = (g >= lo) & (g >= b0) & (g < bL)
                seg = jnp.int32(0)
                for j in range(1, segs_per_w):
                    seg = seg + (g >= bounds[j]).astype(jnp.int32)
                base = jnp.where(valid, seg, segs_per_w) * H
                for l in range(lanes_per_row):
                    plsc.addupdate(acc.at[pl.ds(base + l * 16, 16)],
                                   buf[r, pl.ds(l * 16, 16)])
                return carry

            return lax.fori_loop(0, _CH, row_body, 0)

        @pl.when(nch > 0)
        def _():
            pltpu.async_copy(emb_hbm.at[pl.ds(off(0), _CH)], buf0, sem0)

        def pair_body(p, carry):
            k0 = 2 * p
            pltpu.make_async_copy(
                emb_hbm.at[pl.ds(0, _CH)], buf0, sem0).wait()

            @pl.when(k0 + 1 < nch)
            def _():
                pltpu.async_copy(
                    emb_hbm.at[pl.ds(off(k0 + 1), _CH)], buf1, sem1)

            accum(buf0, k0)

            @pl.when(k0 + 1 < nch)
            def _():
                pltpu.make_async_copy(
                    emb_hbm.at[pl.ds(0, _CH)], buf1, sem1).wait()

                @pl.when(k0 + 2 < nch)
                def _():
                    pltpu.async_copy(
                        emb_hbm.at[pl.ds(off(k0 + 2), _CH)], buf0, sem0)

                accum(buf1, k0 + 1)

            return carry

        lax.fori_loop(0, (nch + 1) // 2, pair_body, 0)

        pltpu.sync_copy(
            acc.at[pl.ds(0, segs_per_w * H)],
            out_hbm.at[pl.ds(wid * segs_per_w * H, segs_per_w * H)])

    return seg_sum


def _mlp_body(s_ref, c_ref, w1_ref, b1_ref, w2_ref, b2_ref, w3_ref, b3_ref,
              o_ref):
    cnt = jnp.maximum(c_ref[...], 1.0)
    x = s_ref[...] / cnt
    h = jnp.dot(x, w1_ref[...], preferred_element_type=jnp.float32)
    h = jnp.maximum(h + b1_ref[...], 0.0)
    h = jnp.dot(h, w2_ref[...], preferred_element_type=jnp.float32)
    h = jnp.maximum(h + b2_ref[...], 0.0)
    o = jnp.sum(h * w3_ref[...], axis=1, keepdims=True) + b3_ref[...]
    o_ref[...] = 1.0 / (1.0 + jnp.exp(-o))


def kernel(node_embedding, W1, b1, W2, b2, W3, b3, node_type, num_variable):
    N, H = node_embedding.shape
    B = num_variable.shape[0]
    segs_per_w = B // _NW

    offsets = jnp.concatenate(
        [jnp.zeros((1,), jnp.int32), jnp.cumsum(num_variable, dtype=jnp.int32)])
    idx = segs_per_w * jnp.arange(_NW)[:, None] + jnp.arange(segs_per_w + 1)
    tbl = jnp.pad(offsets[idx], ((0, 0), (0, 16 - (segs_per_w + 1))))

    sums = _make_seg_sum(N, H, B)(
        node_embedding, tbl.reshape(-1).astype(jnp.int32))
    sums = sums.reshape(B, H)

    out = pl.pallas_call(
        _mlp_body,
        out_shape=jax.ShapeDtypeStruct((B, 1), jnp.float32),
    )(
        sums,
        num_variable.astype(jnp.float32).reshape(B, 1),
        W1, b1.reshape(1, H),
        W2, b2.reshape(1, H),
        W3.reshape(1, H),
        b3.reshape(1, 1),
    )
    return out.reshape(B)


# speedup vs baseline: 1.8818x; 1.8818x over previous
import functools

import jax
import jax.numpy as jnp
from jax import lax
from jax.experimental import pallas as pl
from jax.experimental.pallas import tpu as pltpu
from jax.experimental.pallas import tpu_sc as plsc

_NC = 2
_NS = 16
_NW = _NC * _NS
_LANES = 16
_CH = 128


def _make_seg_sum(N, H, B):
    segs_per_w = B // _NW
    lanes_per_row = H // _LANES
    mesh = plsc.VectorSubcoreMesh(core_axis_name="c", subcore_axis_name="s")

    @functools.partial(
        pl.kernel,
        mesh=mesh,
        out_type=jax.ShapeDtypeStruct((B * H,), jnp.float32),
        scratch_types=[
            pltpu.VMEM((16,), jnp.int32),
            pltpu.VMEM((_CH, H), jnp.float32),
            pltpu.VMEM((_CH, H), jnp.float32),
            pltpu.VMEM((segs_per_w * H,), jnp.float32),
            pltpu.SemaphoreType.DMA,
            pltpu.SemaphoreType.DMA,
        ],
    )
    def seg_sum(emb_hbm, tbl_hbm, out_hbm, tblv, buf0, buf1, outv, sem0, sem1):
        wid = lax.axis_index("s") * _NC + lax.axis_index("c")
        pltpu.sync_copy(tbl_hbm.at[pl.ds(wid * 16, 16)], tblv)
        vec = tblv[...]
        bounds = [vec[j] for j in range(segs_per_w + 1)]

        def start(k, buf, sem, a):
            o = pl.multiple_of(jnp.minimum(a + k * _CH, N - _CH), 8)
            pltpu.async_copy(emb_hbm.at[pl.ds(o, _CH)], buf, sem)

        def wait(buf, sem):
            pltpu.make_async_copy(emb_hbm.at[pl.ds(0, _CH)], buf, sem).wait()

        for j in range(segs_per_w):
            s = bounds[j]
            e = bounds[j + 1]
            a = (s // 8) * 8
            nch = (e - a + _CH - 1) // _CH
            npairs = (nch + 1) // 2

            def accum(buf, k, accs, s=s, e=e, a=a):
                lo = a + k * _CH
                o = jnp.minimum(lo, N - _CH)

                def row_body(r, accs):
                    g = o + r
                    valid = (g >= lo) & (g >= s) & (g < e)
                    mv = jnp.full((16,), jnp.where(valid, 1.0, 0.0),
                                  dtype=jnp.float32)
                    return tuple(
                        accs[l] + buf[r, pl.ds(l * 16, 16)] * mv
                        for l in range(lanes_per_row)
                    )

                return lax.fori_loop(0, _CH, row_body, accs)

            start(0, buf0, sem0, a)
            start(1, buf1, sem1, a)

            def pair_body(p, accs, a=a):
                wait(buf0, sem0)
                accs = accum(buf0, 2 * p, accs)
                start(2 * p + 2, buf0, sem0, a)
                wait(buf1, sem1)
                accs = accum(buf1, 2 * p + 1, accs)
                start(2 * p + 3, buf1, sem1, a)
                return accs

            accs = lax.fori_loop(
                0, jnp.maximum(npairs - 1, 0), pair_body,
                tuple(jnp.zeros((16,), jnp.float32)
                      for _ in range(lanes_per_row)))
            wait(buf0, sem0)
            accs = accum(buf0, 2 * npairs - 2, accs)
            wait(buf1, sem1)
            accs = accum(buf1, 2 * npairs - 1, accs)

            for l in range(lanes_per_row):
                outv[pl.ds(j * H + l * 16, 16)] = accs[l]

        pltpu.sync_copy(
            outv, out_hbm.at[pl.ds(wid * segs_per_w * H, segs_per_w * H)])

    return seg_sum


def _mlp_body(s_ref, c_ref, w1_ref, b1_ref, w2_ref, b2_ref, w3_ref, b3_ref,
              o_ref):
    cnt = jnp.maximum(c_ref[...], 1.0)
    x = s_ref[...] / cnt
    h = jnp.dot(x, w1_ref[...], preferred_element_type=jnp.float32)
    h = jnp.maximum(h + b1_ref[...], 0.0)
    h = jnp.dot(h, w2_ref[...], preferred_element_type=jnp.float32)
    h = jnp.maximum(h + b2_ref[...], 0.0)
    o = jnp.sum(h * w3_ref[...], axis=1, keepdims=True) + b3_ref[...]
    o_ref[...] = 1.0 / (1.0 + jnp.exp(-o))


def kernel(node_embedding, W1, b1, W2, b2, W3, b3, node_type, num_variable):
    N, H = node_embedding.shape
    B = num_variable.shape[0]
    segs_per_w = B // _NW

    offsets = jnp.concatenate(
        [jnp.zeros((1,), jnp.int32), jnp.cumsum(num_variable, dtype=jnp.int32)])
    idx = segs_per_w * jnp.arange(_NW)[:, None] + jnp.arange(segs_per_w + 1)
    tbl = jnp.pad(offsets[idx], ((0, 0), (0, 16 - (segs_per_w + 1))))

    sums = _make_seg_sum(N, H, B)(
        node_embedding, tbl.reshape(-1).astype(jnp.int32))
    sums = sums.reshape(B, H)

    out = pl.pallas_call(
        _mlp_body,
        out_shape=jax.ShapeDtypeStruct((B, 1), jnp.float32),
    )(
        sums,
        num_variable.astype(jnp.float32).reshape(B, 1),
        W1, b1.reshape(1, H),
        W2, b2.reshape(1, H),
        W3.reshape(1, H),
        b3.reshape(1, 1),
    )
    return out.reshape(B)
